# Initial kernel scaffold; baseline (speedup 1.0000x reference)
import jax
import jax.numpy as jnp
from jax.experimental import pallas as pl


def kernel(scores, g):
    raise NotImplementedError("write your pallas kernel here")


# speedup vs baseline: 3.6909x; 3.6909x over previous
import functools

import jax
import jax.numpy as jnp
import numpy as np
from jax.experimental import pallas as pl

_EPSILON = float(np.finfo(np.float32).tiny)
_K = 8


def _body(s_ref, g_ref, o_ref):
    s = s_ref[...] + g_ref[...]
    m = jnp.max(s, axis=1, keepdims=True)
    u = jnp.exp(s - m)
    khot = jnp.zeros_like(u)
    for t in range(_K):
        denom = jnp.sum(u, axis=1, keepdims=True)
        onehot = u * (1.0 / denom)
        khot = khot + onehot
        if t < _K - 1:
            u = u * jnp.maximum(1.0 - onehot, _EPSILON)
    o_ref[...] = khot


@jax.jit
def kernel(scores, g):
    n_rows, n_cols = scores.shape
    block_rows = 8
    grid = (n_rows // block_rows,)
    spec = pl.BlockSpec((block_rows, n_cols), lambda i: (i, 0))
    return pl.pallas_call(
        _body,
        grid=grid,
        in_specs=[spec, spec],
        out_specs=spec,
        out_shape=jax.ShapeDtypeStruct((n_rows, n_cols), jnp.float32),
    )(scores, g)


# speedup vs baseline: 5.0234x; 1.3610x over previous
import functools

import jax
import jax.numpy as jnp
import numpy as np
from jax.experimental import pallas as pl

_EPSILON = float(np.finfo(np.float32).tiny)
_K = 8


def _body(s_ref, g_ref, o_ref):
    s = s_ref[...] + g_ref[...]
    m = jnp.max(s, axis=1, keepdims=True)
    u = jnp.exp(s - m)
    khot = None
    for t in range(_K):
        denom = jnp.sum(u, axis=1, keepdims=True)
        onehot = u * (1.0 / denom)
        khot = onehot if khot is None else khot + onehot
        if t < _K - 1:
            u = u - u * onehot
    o_ref[...] = khot


@jax.jit
def kernel(scores, g):
    n_rows, n_cols = scores.shape
    block_rows = 32
    grid = (n_rows // block_rows,)
    spec = pl.BlockSpec((block_rows, n_cols), lambda i: (i, 0))
    return pl.pallas_call(
        _body,
        grid=grid,
        in_specs=[spec, spec],
        out_specs=spec,
        out_shape=jax.ShapeDtypeStruct((n_rows, n_cols), jnp.float32),
    )(scores, g)
